# baseline (speedup 1.0000x reference)
import functools

import jax
import jax.numpy as jnp
from jax import lax
from jax.experimental import pallas as pl
from jax.experimental.pallas import tpu as pltpu
from jax.experimental.pallas import tpu_sc as plsc

_NC = 2
_NS = 16
_NW = _NC * _NS

_S = 50
_B = 16384
_D = 32
_ST_FULL = _S // 8
_S_TAIL = _S - 8 * _ST_FULL
_BT = _B // 128
_FULL_TILES = _ST_FULL * _BT
_TILES_PER_W = (_ST_FULL + 1) * _BT // _NW
_FULL_PER_W = _FULL_TILES // _NW
_TAIL_PER_W = _TILES_PER_W - _FULL_PER_W
_NCH = 8 * _FULL_PER_W + _S_TAIL * _TAIL_PER_W
_NFCH = 8 * _FULL_PER_W


@functools.lru_cache(maxsize=None)
def _make_repack(V, D):
    n_full = V // 128
    base_per_w = n_full // _NW
    extra = n_full - base_per_w * _NW
    n_pairs = base_per_w // 2
    rem_rows = V - 128 * n_full
    mesh = plsc.VectorSubcoreMesh(core_axis_name="c", subcore_axis_name="s")

    @functools.partial(
        pl.kernel,
        out_type=jax.ShapeDtypeStruct((V * D // 128, 128), jnp.float32),
        mesh=mesh,
        scratch_types=[
            [pltpu.VMEM((D, 128), jnp.float32) for _ in range(2)],
            [pltpu.VMEM((D, 128), jnp.float32) for _ in range(2)],
            pltpu.SemaphoreType.DMA,
            pltpu.SemaphoreType.DMA,
        ],
        compiler_params=pltpu.CompilerParams(
            use_tc_tiling_on_sc=True, needs_layout_passes=False),
    )
    def k(tabT, tailT, tabL, srcs, dsts, isem, osem):
        wid = lax.axis_index("s") * _NC + lax.axis_index("c")

        def fire_src(rt, b):
            pltpu.async_copy(
                tabT.at[:, pl.ds(128 * rt, 128)], srcs[b], isem)

        def wait_src(b):
            pltpu.make_async_copy(
                tabT.at[:, pl.ds(0, 128)], srcs[b], isem).wait()

        def wait_dst(b, qrows=D):
            pltpu.make_async_copy(
                dsts[b].at[pl.ds(0, qrows)],
                tabL.at[pl.ds(0, qrows)], osem).wait()

        def transpose(b, qrows=D):
            for c0 in range(8):
                dvec = jax.lax.broadcasted_iota(
                    jnp.int32, (16,), 0) + 16 * (c0 & 1)
                j = c0 >> 1
                for qq in range(qrows):
                    rvec = jnp.full((16,), 4 * qq + j, jnp.int32)
                    dsts[b][qq, pl.ds(16 * c0, 16)] = plsc.load_gather(
                        srcs[b], [dvec, rvec])

        def fire_dst(rt, b, qrows=D):
            pltpu.async_copy(
                dsts[b].at[pl.ds(0, qrows)],
                tabL.at[pl.ds(qrows * rt, qrows)], osem)

        def step(i, b, first, last):
            rt = wid + _NW * i
            if not last:
                pltpu.async_copy(
                    tabT.at[:, pl.ds(128 * (rt + _NW), 128)],
                    srcs[1 - b], isem)
            wait_src(b)
            if not first:
                wait_dst(b)
            transpose(b)
            fire_dst(rt, b)

        fire_src(wid, 0)
        step(0, 0, True, False)
        step(1, 1, True, False)

        def pair(p, carry):
            step(2 * p, 0, False, False)
            step(2 * p + 1, 1, False, False)
            return carry

        lax.fori_loop(1, n_pairs - 1, pair, 0)
        step(base_per_w - 2, 0, False, False)
        step(base_per_w - 1, 1, False, True)
        wait_dst(0)
        wait_dst(1)

        @pl.when(wid < extra)
        def _():
            rt = _NW * base_per_w + wid
            fire_src(rt, 0)
            wait_src(0)
            transpose(0)
            fire_dst(rt, 0)
            wait_dst(0)

        @pl.when(wid == extra)
        def _():
            qr = rem_rows // 4
            pltpu.async_copy(tailT, srcs[0], isem)
            wait_src(0)
            transpose(0, qrows=qr)
            pltpu.async_copy(
                dsts[0].at[pl.ds(0, qr)],
                tabL.at[pl.ds(D * n_full, qr)], osem)
            wait_dst(0, qrows=qr)

    return k


@functools.lru_cache(maxsize=None)
def _make_lookup(Vq):
    mesh = plsc.VectorSubcoreMesh(core_axis_name="c", subcore_axis_name="s")

    @functools.partial(
        pl.kernel,
        out_type=jax.ShapeDtypeStruct((_S, _D, _B), jnp.float32),
        mesh=mesh,
        scratch_types=[
            pltpu.VMEM((_TILES_PER_W, 8, 128), jnp.int32),
            pltpu.VMEM((4, 128), jnp.int32),
            pltpu.VMEM((4, 128, 128), jnp.float32),
            [pltpu.VMEM((_D, 128), jnp.float32) for _ in range(2)],
            pltpu.SemaphoreType.DMA,
            pltpu.SemaphoreType.DMA,
            pltpu.SemaphoreType.DMA,
        ],
        compiler_params=pltpu.CompilerParams(
            use_tc_tiling_on_sc=True, needs_layout_passes=False),
    )
    def k(idxT, tabL, outT, idx_all, qring, rows, ostage, isem, gsem, wsem):
        wid = lax.axis_index("s") * _NC + lax.axis_index("c")

        def stage_full(kt, carry):
            t = wid + _NW * kt
            st = jax.lax.shift_right_logical(t, 7)
            bt = jax.lax.bitwise_and(t, _BT - 1)
            pltpu.async_copy(
                idxT.at[pl.ds(8 * st, 8), pl.ds(128 * bt, 128)],
                idx_all.at[kt], isem)
            return carry

        lax.fori_loop(0, _FULL_PER_W, stage_full, 0)

        def stage_tail(i, carry):
            bt = wid + _NW * i
            pltpu.async_copy(
                idxT.at[pl.ds(8 * _ST_FULL, _S_TAIL), pl.ds(128 * bt, 128)],
                idx_all.at[_FULL_PER_W + i, pl.ds(0, _S_TAIL)], isem)
            return carry

        lax.fori_loop(0, _TAIL_PER_W, stage_tail, 0)

        def drain_full(i, carry):
            pltpu.make_async_copy(
                idxT.at[pl.ds(0, 8), pl.ds(0, 128)],
                idx_all.at[0], isem).wait()
            return carry

        lax.fori_loop(0, _FULL_PER_W, drain_full, 0)

        def drain_tail(i, carry):
            pltpu.make_async_copy(
                idxT.at[pl.ds(0, _S_TAIL), pl.ds(0, 128)],
                idx_all.at[0, pl.ds(0, _S_TAIL)], isem).wait()
            return carry

        lax.fori_loop(0, _TAIL_PER_W, drain_tail, 0)

        def chunk_coords(c):
            full = c < _NFCH
            k_f = jax.lax.shift_right_logical(c, 3)
            k_t = _FULL_PER_W + jax.lax.shift_right_logical(c - _NFCH, 1)
            kt = jnp.where(full, k_f, k_t)
            s_loc = jnp.where(full, jax.lax.bitwise_and(c, 7),
                              jax.lax.bitwise_and(c - _NFCH, 1))
            t = wid + _NW * kt
            st_f = jax.lax.shift_right_logical(t, 7)
            st = jnp.where(full, st_f, _ST_FULL)
            bt = jnp.where(full, jax.lax.bitwise_and(t, _BT - 1),
                           wid + _NW * (kt - _FULL_PER_W))
            return kt, s_loc, 8 * st + s_loc, bt

        def prep_and_fire(c, slot):
            kt, s_loc, _, _ = chunk_coords(c)
            for g in range(8):
                v = idx_all[kt, s_loc, pl.ds(16 * g, 16)]
                qring[slot, pl.ds(16 * g, 16)] = (
                    jax.lax.shift_right_logical(v, 2))
            pltpu.async_copy(tabL.at[qring.at[slot]], rows.at[slot], gsem)

        def wait_gather(slot):
            pltpu.make_async_copy(
                tabL.at[qring.at[slot]], rows.at[slot], gsem).wait()

        def select_and_out(c, slot, oslot):
            kt, s_loc, s_glob, bt = chunk_coords(c)
            obuf = ostage[oslot]
            for bg in range(8):
                rvec = jax.lax.broadcasted_iota(
                    jnp.int32, (16,), 0) + (16 * bg)
                cvec = jax.lax.shift_left(jax.lax.bitwise_and(
                    idx_all[kt, s_loc, pl.ds(16 * bg, 16)], 3), 5)
                for d in range(_D):
                    obuf[d, pl.ds(16 * bg, 16)] = plsc.load_gather(
                        rows.at[slot], [rvec, cvec + d])
            pltpu.async_copy(
                obuf, outT.at[s_glob, :, pl.ds(128 * bt, 128)], wsem)

        def wait_out(oslot):
            pltpu.make_async_copy(
                ostage[oslot], outT.at[0, :, pl.ds(0, 128)], wsem).wait()

        prep_and_fire(0, 0)
        prep_and_fire(1, 1)

        def group(G, carry):
            c0 = 4 * G
            for j in range(4):
                c = c0 + j

                @pl.when(c + 2 < _NCH)
                def _():
                    prep_and_fire(c + 2, (j + 2) % 4)

                wait_gather(j)

                @pl.when(c >= 2)
                def _():
                    wait_out(j % 2)

                select_and_out(c, j, j % 2)
            return carry

        lax.fori_loop(0, _NCH // 4, group, 0)
        wait_out(0)
        wait_out(1)

    return k


def kernel(input, embedding_weight):
    V, D = embedding_weight.shape
    rem = V % 128
    tailT = jnp.pad(embedding_weight[V - rem:],
                    ((0, 128 - rem), (0, 0))).T
    tabL = _make_repack(V, D)(embedding_weight.T, tailT)
    outT = _make_lookup(tabL.shape[0])(input.T, tabL)
    return outT.transpose(2, 0, 1)


# speedup vs baseline: 1.2942x; 1.2942x over previous
import functools

import jax
import jax.numpy as jnp
from jax import lax
from jax.experimental import pallas as pl
from jax.experimental.pallas import tpu as pltpu
from jax.experimental.pallas import tpu_sc as plsc

_NC = 2
_NS = 16
_NW = _NC * _NS

_S = 50
_B = 16384
_D = 32
_ST_FULL = _S // 8
_S_TAIL = _S - 8 * _ST_FULL
_BT = _B // 128
_FULL_TILES = _ST_FULL * _BT
_TILES_PER_W = (_ST_FULL + 1) * _BT // _NW
_FULL_PER_W = _FULL_TILES // _NW
_TAIL_PER_W = _TILES_PER_W - _FULL_PER_W
_NCH = 8 * _FULL_PER_W + _S_TAIL * _TAIL_PER_W
_NFCH = 8 * _FULL_PER_W


@functools.lru_cache(maxsize=None)
def _make_lookup(Vq):
    mesh = plsc.VectorSubcoreMesh(core_axis_name="c", subcore_axis_name="s")

    @functools.partial(
        pl.kernel,
        out_type=jax.ShapeDtypeStruct((_S, _D, _B), jnp.float32),
        mesh=mesh,
        scratch_types=[
            pltpu.VMEM((_TILES_PER_W, 8, 128), jnp.int32),
            pltpu.VMEM((4, 128), jnp.int32),
            pltpu.VMEM((4, 128, 128), jnp.float32),
            [pltpu.VMEM((_D, 128), jnp.float32) for _ in range(4)],
            pltpu.SemaphoreType.DMA,
            pltpu.SemaphoreType.DMA,
            pltpu.SemaphoreType.DMA,
        ],
        compiler_params=pltpu.CompilerParams(
            use_tc_tiling_on_sc=True, needs_layout_passes=False),
    )
    def k(idxT, tabL, outT, idx_all, qring, rows, ostage, isem, gsem, wsem):
        wid = lax.axis_index("s") * _NC + lax.axis_index("c")

        def stage_full(kt, carry):
            t = wid + _NW * kt
            st = jax.lax.shift_right_logical(t, 7)
            bt = jax.lax.bitwise_and(t, _BT - 1)
            pltpu.async_copy(
                idxT.at[pl.ds(8 * st, 8), pl.ds(128 * bt, 128)],
                idx_all.at[kt], isem)
            return carry

        lax.fori_loop(0, _FULL_PER_W, stage_full, 0)

        def stage_tail(i, carry):
            bt = wid + _NW * i
            pltpu.async_copy(
                idxT.at[pl.ds(8 * _ST_FULL, _S_TAIL), pl.ds(128 * bt, 128)],
                idx_all.at[_FULL_PER_W + i, pl.ds(0, _S_TAIL)], isem)
            return carry

        lax.fori_loop(0, _TAIL_PER_W, stage_tail, 0)

        def drain_full(i, carry):
            pltpu.make_async_copy(
                idxT.at[pl.ds(0, 8), pl.ds(0, 128)],
                idx_all.at[0], isem).wait()
            return carry

        lax.fori_loop(0, _FULL_PER_W, drain_full, 0)

        def drain_tail(i, carry):
            pltpu.make_async_copy(
                idxT.at[pl.ds(0, _S_TAIL), pl.ds(0, 128)],
                idx_all.at[0, pl.ds(0, _S_TAIL)], isem).wait()
            return carry

        lax.fori_loop(0, _TAIL_PER_W, drain_tail, 0)

        def chunk_coords(c):
            full = c < _NFCH
            k_f = jax.lax.shift_right_logical(c, 3)
            k_t = _FULL_PER_W + jax.lax.shift_right_logical(c - _NFCH, 1)
            kt = jnp.where(full, k_f, k_t)
            s_loc = jnp.where(full, jax.lax.bitwise_and(c, 7),
                              jax.lax.bitwise_and(c - _NFCH, 1))
            t = wid + _NW * kt
            st_f = jax.lax.shift_right_logical(t, 7)
            st = jnp.where(full, st_f, _ST_FULL)
            bt = jnp.where(full, jax.lax.bitwise_and(t, _BT - 1),
                           wid + _NW * (kt - _FULL_PER_W))
            return kt, s_loc, 8 * st + s_loc, bt

        def prep_and_fire(c, slot):
            kt, s_loc, _, _ = chunk_coords(c)
            for g in range(8):
                v = idx_all[kt, s_loc, pl.ds(16 * g, 16)]
                qring[slot, pl.ds(16 * g, 16)] = (
                    jax.lax.shift_right_logical(v, 2))
            pltpu.async_copy(tabL.at[qring.at[slot]], rows.at[slot], gsem)

        def wait_gather(slot):
            pltpu.make_async_copy(
                tabL.at[qring.at[slot]], rows.at[slot], gsem).wait()

        def select_and_out(c, slot, oslot):
            kt, s_loc, s_glob, bt = chunk_coords(c)
            obuf = ostage[oslot]
            for bg in range(8):
                rvec = jax.lax.broadcasted_iota(
                    jnp.int32, (16,), 0) + (16 * bg)
                cvec = jax.lax.shift_left(jax.lax.bitwise_and(
                    idx_all[kt, s_loc, pl.ds(16 * bg, 16)], 3), 5)
                for d in range(_D):
                    obuf[d, pl.ds(16 * bg, 16)] = plsc.load_gather(
                        rows.at[slot], [rvec, cvec + d])
            pltpu.async_copy(
                obuf, outT.at[s_glob, :, pl.ds(128 * bt, 128)], wsem)

        def wait_out(oslot):
            pltpu.make_async_copy(
                ostage[oslot], outT.at[0, :, pl.ds(0, 128)], wsem).wait()

        prep_and_fire(0, 0)
        prep_and_fire(1, 1)
        prep_and_fire(2, 2)

        def group(G, carry):
            c0 = 4 * G
            for j in range(4):
                c = c0 + j

                @pl.when(c + 3 < _NCH)
                def _():
                    prep_and_fire(c + 3, (j + 3) % 4)

                wait_gather(j)

                @pl.when(c >= 4)
                def _():
                    wait_out(j)

                select_and_out(c, j, j)
            return carry

        lax.fori_loop(0, _NCH // 4, group, 0)
        for j in range(4):
            wait_out(j)

    return k


def kernel(input, embedding_weight):
    V, D = embedding_weight.shape
    idxT = input.T
    tabL = embedding_weight.reshape(-1, 128)
    outT = _make_lookup(tabL.shape[0])(idxT, tabL)
    return outT.transpose(2, 0, 1)


# speedup vs baseline: 1.3039x; 1.0075x over previous
import functools

import jax
import jax.numpy as jnp
from jax import lax
from jax.experimental import pallas as pl
from jax.experimental.pallas import tpu as pltpu
from jax.experimental.pallas import tpu_sc as plsc

_NC = 2
_NS = 16
_NW = _NC * _NS

_S = 50
_B = 16384
_D = 32
_ST_FULL = _S // 8
_S_TAIL = _S - 8 * _ST_FULL
_BT = _B // 128
_FULL_TILES = _ST_FULL * _BT
_TILES_PER_W = (_ST_FULL + 1) * _BT // _NW
_FULL_PER_W = _FULL_TILES // _NW
_TAIL_PER_W = _TILES_PER_W - _FULL_PER_W
_NCH = 8 * _FULL_PER_W + _S_TAIL * _TAIL_PER_W
_NFCH = 8 * _FULL_PER_W


@functools.lru_cache(maxsize=None)
def _make_lookup(Vq):
    mesh = plsc.VectorSubcoreMesh(core_axis_name="c", subcore_axis_name="s")

    @functools.partial(
        pl.kernel,
        out_type=jax.ShapeDtypeStruct((_S, _D, _B), jnp.float32),
        mesh=mesh,
        scratch_types=[
            pltpu.VMEM((_TILES_PER_W, 8, 128), jnp.int32),
            pltpu.VMEM((4, 128), jnp.int32),
            pltpu.VMEM((4, 128, 128), jnp.float32),
            [pltpu.VMEM((_D, 128), jnp.float32) for _ in range(2)],
            pltpu.SemaphoreType.DMA,
            pltpu.SemaphoreType.DMA,
            pltpu.SemaphoreType.DMA,
        ],
        compiler_params=pltpu.CompilerParams(
            use_tc_tiling_on_sc=True, needs_layout_passes=False),
    )
    def k(idxT, tabL, outT, idx_all, qring, rows, ostage, isem, gsem, wsem):
        wid = lax.axis_index("s") * _NC + lax.axis_index("c")

        def stage_full(kt, carry):
            t = wid + _NW * kt
            st = jax.lax.shift_right_logical(t, 7)
            bt = jax.lax.bitwise_and(t, _BT - 1)
            pltpu.async_copy(
                idxT.at[pl.ds(8 * st, 8), pl.ds(128 * bt, 128)],
                idx_all.at[kt], isem)
            return carry

        lax.fori_loop(0, _FULL_PER_W, stage_full, 0)

        def stage_tail(i, carry):
            bt = wid + _NW * i
            pltpu.async_copy(
                idxT.at[pl.ds(8 * _ST_FULL, _S_TAIL), pl.ds(128 * bt, 128)],
                idx_all.at[_FULL_PER_W + i, pl.ds(0, _S_TAIL)], isem)
            return carry

        lax.fori_loop(0, _TAIL_PER_W, stage_tail, 0)

        def drain_full(i, carry):
            pltpu.make_async_copy(
                idxT.at[pl.ds(0, 8), pl.ds(0, 128)],
                idx_all.at[0], isem).wait()
            return carry

        lax.fori_loop(0, _FULL_PER_W, drain_full, 0)

        def drain_tail(i, carry):
            pltpu.make_async_copy(
                idxT.at[pl.ds(0, _S_TAIL), pl.ds(0, 128)],
                idx_all.at[0, pl.ds(0, _S_TAIL)], isem).wait()
            return carry

        lax.fori_loop(0, _TAIL_PER_W, drain_tail, 0)

        def chunk_coords(c):
            full = c < _NFCH
            k_f = jax.lax.shift_right_logical(c, 3)
            k_t = _FULL_PER_W + jax.lax.shift_right_logical(c - _NFCH, 1)
            kt = jnp.where(full, k_f, k_t)
            s_loc = jnp.where(full, jax.lax.bitwise_and(c, 7),
                              jax.lax.bitwise_and(c - _NFCH, 1))
            t = wid + _NW * kt
            st_f = jax.lax.shift_right_logical(t, 7)
            st = jnp.where(full, st_f, _ST_FULL)
            bt = jnp.where(full, jax.lax.bitwise_and(t, _BT - 1),
                           wid + _NW * (kt - _FULL_PER_W))
            return kt, s_loc, 8 * st + s_loc, bt

        def prep_and_fire(c, slot):
            kt, s_loc, _, _ = chunk_coords(c)
            for g in range(8):
                v = idx_all[kt, s_loc, pl.ds(16 * g, 16)]
                qring[slot, pl.ds(16 * g, 16)] = (
                    jax.lax.shift_right_logical(v, 2))
            pltpu.async_copy(tabL.at[qring.at[slot]], rows.at[slot], gsem)

        def wait_gather(slot):
            pltpu.make_async_copy(
                tabL.at[qring.at[slot]], rows.at[slot], gsem).wait()

        def select_and_out(c, slot, oslot):
            kt, s_loc, s_glob, bt = chunk_coords(c)
            obuf = ostage[oslot]
            for bg in range(8):
                rvec = jax.lax.broadcasted_iota(
                    jnp.int32, (16,), 0) + (16 * bg)
                cvec = jax.lax.shift_left(jax.lax.bitwise_and(
                    idx_all[kt, s_loc, pl.ds(16 * bg, 16)], 3), 5)
                for d in range(_D):
                    obuf[d, pl.ds(16 * bg, 16)] = plsc.load_gather(
                        rows.at[slot], [rvec, cvec + d])
            pltpu.async_copy(
                obuf, outT.at[s_glob, :, pl.ds(128 * bt, 128)], wsem)

        def wait_out(oslot):
            pltpu.make_async_copy(
                ostage[oslot], outT.at[0, :, pl.ds(0, 128)], wsem).wait()

        prep_and_fire(0, 0)
        prep_and_fire(1, 1)

        def group(G, carry):
            c0 = 4 * G
            for j in range(4):
                c = c0 + j

                @pl.when(c + 2 < _NCH)
                def _():
                    prep_and_fire(c + 2, (j + 2) % 4)

                wait_gather(j)

                @pl.when(c >= 2)
                def _():
                    wait_out(j % 2)

                select_and_out(c, j, j % 2)
            return carry

        lax.fori_loop(0, _NCH // 4, group, 0)
        wait_out(0)
        wait_out(1)

    return k


def kernel(input, embedding_weight):
    V, D = embedding_weight.shape
    idxT = input.T
    tabL = embedding_weight.reshape(-1, 128)
    outT = _make_lookup(tabL.shape[0])(idxT, tabL)
    return outT.transpose(2, 0, 1)
